# Initial kernel scaffold; baseline (speedup 1.0000x reference)
import jax
import jax.numpy as jnp
from jax.experimental import pallas as pl


def kernel(indices, pe):
    raise NotImplementedError("write your pallas kernel here")


# speedup vs baseline: 2.9147x; 2.9147x over previous
import jax
import jax.numpy as jnp
from jax import lax
from jax.experimental import pallas as pl
from jax.experimental.pallas import tpu as pltpu
from jax.experimental.pallas import tpu_sc as plsc

_D = 128
_NW = 32
_CHUNK = 128
_MESH = plsc.VectorSubcoreMesh(core_axis_name="c", subcore_axis_name="s")


def _body(idx_hbm, pe_hbm, out_hbm, idx_v, rows_v, sem):
    wid = lax.axis_index("s") * 2 + lax.axis_index("c")
    n_chunks = idx_v.shape[0]
    pltpu.sync_copy(idx_hbm.at[wid], idx_v)

    def step(j, carry):
        pltpu.async_copy(pe_hbm.at[idx_v.at[j]], rows_v, sem).wait()
        pltpu.sync_copy(
            rows_v, out_hbm.at[pl.ds((wid * n_chunks + j) * _CHUNK, _CHUNK)])
        return carry

    lax.fori_loop(0, n_chunks, step, 0)


def kernel(indices, pe):
    b, l = indices.shape
    total = b * l
    n_chunks = total // (_NW * _CHUNK)
    idx = indices.reshape(_NW, n_chunks, _CHUNK)
    out_flat = pl.kernel(
        _body,
        out_type=jax.ShapeDtypeStruct((total, _D), jnp.float32),
        mesh=_MESH,
        scratch_types=[
            pltpu.VMEM((n_chunks, _CHUNK), jnp.int32),
            pltpu.VMEM((_CHUNK, _D), jnp.float32),
            pltpu.SemaphoreType.DMA,
        ],
    )(idx, pe)
    return out_flat.reshape(b, l, _D)


# speedup vs baseline: 3.2232x; 1.1058x over previous
import jax
import jax.numpy as jnp
from jax import lax
from jax.experimental import pallas as pl
from jax.experimental.pallas import tpu as pltpu
from jax.experimental.pallas import tpu_sc as plsc

_D = 128
_NW = 32
_CHUNK = 128
_NBUF = 5
_MESH = plsc.VectorSubcoreMesh(core_axis_name="c", subcore_axis_name="s")


def _body(idx_hbm, pe_hbm, out_hbm, idx_v, rows_v, gsem, wsem):
    wid = lax.axis_index("s") * 2 + lax.axis_index("c")
    n_chunks = idx_v.shape[0]
    n_groups = n_chunks // _NBUF
    pltpu.sync_copy(idx_hbm.at[wid], idx_v)
    out_base = wid * n_chunks * _CHUNK

    def group(g, carry):
        base = g * _NBUF
        gd = []
        for b in range(_NBUF):
            @pl.when(g > 0)
            def _(b=b):
                pltpu.make_async_copy(
                    rows_v.at[b], out_hbm.at[pl.ds(0, _CHUNK)],
                    wsem.at[b]).wait()
            gd.append(pltpu.async_copy(
                pe_hbm.at[idx_v.at[base + b]], rows_v.at[b], gsem.at[b]))
        for b in range(_NBUF):
            gd[b].wait()
            pltpu.async_copy(
                rows_v.at[b],
                out_hbm.at[pl.ds(out_base + (base + b) * _CHUNK, _CHUNK)],
                wsem.at[b])
        return carry

    lax.fori_loop(0, n_groups, group, 0)
    for b in range(_NBUF):
        pltpu.make_async_copy(
            rows_v.at[b], out_hbm.at[pl.ds(0, _CHUNK)], wsem.at[b]).wait()


def kernel(indices, pe):
    b, l = indices.shape
    total = b * l
    n_chunks = total // (_NW * _CHUNK)
    idx = indices.reshape(_NW, n_chunks, _CHUNK)
    out_flat = pl.kernel(
        _body,
        out_type=jax.ShapeDtypeStruct((total, _D), jnp.float32),
        mesh=_MESH,
        scratch_types=[
            pltpu.VMEM((n_chunks, _CHUNK), jnp.int32),
            pltpu.VMEM((_NBUF, _CHUNK, _D), jnp.float32),
            pltpu.SemaphoreType.DMA((_NBUF,)),
            pltpu.SemaphoreType.DMA((_NBUF,)),
        ],
    )(idx, pe)
    return out_flat.reshape(b, l, _D)


# speedup vs baseline: 3.3690x; 1.0453x over previous
import jax
import jax.numpy as jnp
from jax import lax
from jax.experimental import pallas as pl
from jax.experimental.pallas import tpu as pltpu
from jax.experimental.pallas import tpu_sc as plsc

_D = 128
_NW = 32
_CHUNK = 128
_NBUF = 2
_MESH = plsc.VectorSubcoreMesh(core_axis_name="c", subcore_axis_name="s")


def _body(idx_hbm, pe_hbm, out_hbm, idx_v, rows_v, pe_sh, gsem, wsem):
    sid = lax.axis_index("s")
    wid = sid * 2 + lax.axis_index("c")
    n_chunks = idx_v.shape[0]
    n_groups = n_chunks // _NBUF
    pltpu.sync_copy(idx_hbm.at[wid], idx_v)
    out_base = wid * n_chunks * _CHUNK

    rows_per_sub = pe_sh.shape[0] // 16
    pltpu.sync_copy(pe_hbm.at[pl.ds(sid * rows_per_sub, rows_per_sub)],
                    pe_sh.at[pl.ds(sid * rows_per_sub, rows_per_sub)])
    plsc.subcore_barrier()

    def group(g, carry):
        base = g * _NBUF
        gd = []
        for b in range(_NBUF):
            @pl.when(g > 0)
            def _(b=b):
                pltpu.make_async_copy(
                    rows_v.at[b], out_hbm.at[pl.ds(0, _CHUNK)],
                    wsem.at[b]).wait()
            gd.append(pltpu.async_copy(
                pe_sh.at[idx_v.at[base + b]], rows_v.at[b], gsem.at[b]))
        for b in range(_NBUF):
            gd[b].wait()
            pltpu.async_copy(
                rows_v.at[b],
                out_hbm.at[pl.ds(out_base + (base + b) * _CHUNK, _CHUNK)],
                wsem.at[b])
        return carry

    lax.fori_loop(0, n_groups, group, 0)
    for b in range(_NBUF):
        pltpu.make_async_copy(
            rows_v.at[b], out_hbm.at[pl.ds(0, _CHUNK)], wsem.at[b]).wait()


def kernel(indices, pe):
    b, l = indices.shape
    total = b * l
    n_chunks = total // (_NW * _CHUNK)
    idx = indices.reshape(_NW, n_chunks, _CHUNK)
    out_flat = pl.kernel(
        _body,
        out_type=jax.ShapeDtypeStruct((total, _D), jnp.float32),
        mesh=_MESH,
        scratch_types=[
            pltpu.VMEM((n_chunks, _CHUNK), jnp.int32),
            pltpu.VMEM((_NBUF, _CHUNK, _D), jnp.float32),
            pltpu.VMEM_SHARED(pe.shape, jnp.float32),
            pltpu.SemaphoreType.DMA((_NBUF,)),
            pltpu.SemaphoreType.DMA((_NBUF,)),
        ],
    )(idx, pe)
    return out_flat.reshape(b, l, _D)


# speedup vs baseline: 6.9192x; 2.0537x over previous
import jax
import jax.numpy as jnp
from jax import lax
from jax.experimental import pallas as pl
from jax.experimental.pallas import tpu as pltpu
from jax.experimental.pallas import tpu_sc as plsc

_NW = 32
_WB = 2
_NBUF = 2
_MESH = plsc.VectorSubcoreMesh(core_axis_name="c", subcore_axis_name="s")


def _body(idx_hbm, pe_hbm, out_hbm, idx_v, rows_v, pe_sh, gsem, wsem):
    sid = lax.axis_index("s")
    wid = sid * 2 + lax.axis_index("c")
    b_per_w = idx_v.shape[0]
    n_groups = b_per_w // (_WB * _NBUF)
    pltpu.sync_copy(idx_hbm.at[wid], idx_v)
    out_base = wid * b_per_w

    rows_per_sub = pe_sh.shape[0] // 16
    pltpu.sync_copy(pe_hbm.at[pl.ds(sid * rows_per_sub, rows_per_sub)],
                    pe_sh.at[pl.ds(sid * rows_per_sub, rows_per_sub)])
    plsc.subcore_barrier()

    def group(t, carry):
        for u in range(_NBUF):
            g = t * _NBUF + u

            @pl.when(t > 0)
            def _(u=u):
                pltpu.make_async_copy(
                    rows_v.at[u], out_hbm.at[pl.ds(0, _WB)],
                    wsem.at[u]).wait()

            gd = [pltpu.async_copy(
                      pe_sh.at[idx_v.at[g * _WB + k]],
                      rows_v.at[u].at[k], gsem.at[u])
                  for k in range(_WB)]
            for d in gd:
                d.wait()
            pltpu.async_copy(
                rows_v.at[u],
                out_hbm.at[pl.ds(out_base + g * _WB, _WB)],
                wsem.at[u])
        return carry

    lax.fori_loop(0, n_groups, group, 0)
    for u in range(_NBUF):
        pltpu.make_async_copy(
            rows_v.at[u], out_hbm.at[pl.ds(0, _WB)], wsem.at[u]).wait()


def kernel(indices, pe):
    b, l = indices.shape
    d = pe.shape[1]
    b_per_w = b // _NW
    idx = indices.reshape(_NW, b_per_w, l)
    return pl.kernel(
        _body,
        out_type=jax.ShapeDtypeStruct((b, l, d), jnp.float32),
        mesh=_MESH,
        scratch_types=[
            pltpu.VMEM((b_per_w, l), jnp.int32),
            pltpu.VMEM((_NBUF, _WB, l, d), jnp.float32),
            pltpu.VMEM_SHARED(pe.shape, jnp.float32),
            pltpu.SemaphoreType.DMA((_NBUF,)),
            pltpu.SemaphoreType.DMA((_NBUF,)),
        ],
    )(idx, pe)
